# baseline (speedup 1.0000x reference)
import jax
import jax.numpy as jnp
from jax import lax
from jax.experimental import pallas as pl
from jax.experimental.pallas import tpu as pltpu

K_IN = 3 * 84 * 84
D = 512
KODES = 512
NS = 100
NQ = 300
BK = 2688
KT = (K_IN + BK - 1) // BK

_HI = lax.Precision.HIGHEST


def _dot(a, b, dims):
    return lax.dot_general(a, b, (dims, ((), ())), precision=_HI,
                           preferred_element_type=jnp.float32)


def _dot_bf16(a, b, dims):
    return lax.dot_general(a.astype(jnp.bfloat16), b.astype(jnp.bfloat16),
                           (dims, ((), ())),
                           preferred_element_type=jnp.float32)


def _body(xs_ref, xq_ref, w_ref, b_ref, cb_ref, t_ref, out_ref, accs, accq):
    k = pl.program_id(0)

    @pl.when(k == 0)
    def _init():
        accs[...] = jnp.zeros_like(accs)
        accq[...] = jnp.zeros_like(accq)

    rem = K_IN - k * BK
    colmask = lax.broadcasted_iota(jnp.int32, (1, BK), 1) < rem
    rowmask = lax.broadcasted_iota(jnp.int32, (BK, 1), 0) < rem
    xs = jnp.where(colmask, xs_ref[...], 0.0)
    xq = jnp.where(colmask, xq_ref[...], 0.0)
    w = jnp.where(rowmask, w_ref[...], 0.0)
    accs[...] += _dot_bf16(xs, w, (((1,), (0,))))
    accq[...] += _dot_bf16(xq, w, (((1,), (0,))))

    @pl.when(k == KT - 1)
    def _epilogue():
        bias = b_ref[...]
        cb = cb_ref[...]
        ones = jnp.ones((1, D), jnp.float32)
        cn = _dot(ones, cb * cb, ((1,), (1,)))

        def quantize(z):
            zc = _dot_bf16(z, cb, ((1,), (1,)))
            zn = jnp.sum(z * z, axis=1, keepdims=True)
            dist = zn - 2.0 * zc + cn
            mn = jnp.min(dist, axis=1, keepdims=True)
            ii = lax.broadcasted_iota(jnp.int32, dist.shape, 1)
            idx = jnp.min(jnp.where(dist == mn, ii, KODES), axis=1,
                          keepdims=True)
            return (ii == idx).astype(jnp.float32)

        zs = accs[...] + bias
        zq = accq[...] + bias
        qs = _dot(quantize(zs), cb, ((1,), (0,)))
        qq = _dot(quantize(zq), cb, ((1,), (0,)))

        gi = lax.broadcasted_iota(jnp.int32, (20, NS), 0)
        ci = lax.broadcasted_iota(jnp.int32, (20, NS), 1)
        sel = (ci // 5 == gi).astype(jnp.float32)
        proto = _dot(sel, qs, ((1,), (0,))) / 5.0
        pn = jnp.sqrt(jnp.sum(proto * proto, axis=1, keepdims=True))
        proto_n = proto / jnp.maximum(pn, 1e-12)
        qn = jnp.sqrt(jnp.sum(qq * qq, axis=1, keepdims=True))
        xq_n = qq / jnp.maximum(qn, 1e-12)

        lg = _dot_bf16(xq_n, proto_n, ((1,), (1,))) * t_ref[0, 0]
        for bb in range(4):
            out_ref[bb, :, :] = lg[75 * bb:75 * (bb + 1), 5 * bb:5 * (bb + 1)]


def kernel(x_shot, x_query, enc_W, enc_b, codebook, temp):
    xs = x_shot.reshape(NS, K_IN)
    xq = x_query.reshape(NQ, K_IN)
    b2 = enc_b.reshape(1, D)
    t2 = jnp.asarray(temp, jnp.float32).reshape(1, 1)
    return pl.pallas_call(
        _body,
        grid=(KT,),
        in_specs=[
            pl.BlockSpec((NS, BK), lambda k: (0, k)),
            pl.BlockSpec((NQ, BK), lambda k: (0, k)),
            pl.BlockSpec((BK, D), lambda k: (k, 0)),
            pl.BlockSpec((1, D), lambda k: (0, 0)),
            pl.BlockSpec((KODES, D), lambda k: (0, 0)),
            pl.BlockSpec((1, 1), lambda k: (0, 0)),
        ],
        out_specs=pl.BlockSpec((4, 75, 5), lambda k: (0, 0, 0)),
        out_shape=jax.ShapeDtypeStruct((4, 75, 5), jnp.float32),
        scratch_shapes=[
            pltpu.VMEM((NS, D), jnp.float32),
            pltpu.VMEM((NQ, D), jnp.float32),
        ],
    )(xs, xq, enc_W, b2, codebook, t2)
